# Initial kernel scaffold; baseline (speedup 1.0000x reference)
import jax
import jax.numpy as jnp
from jax.experimental import pallas as pl


def kernel(x, adj, hw1, hb1, hw2, hb2, hg, hbeta, rw1, rb1, rw2, rb2, rg, rbeta, fw, fb, p1w, p1b, p2w, p2b, p3w, p3b):
    raise NotImplementedError("write your pallas kernel here")


# speedup vs baseline: 9.2448x; 9.2448x over previous
import jax
import jax.numpy as jnp
from jax.experimental import pallas as pl

F32 = jnp.float32
EPS = 1e-5


def _build_gather_matrix(adj, n, k):
    def body(adj_ref, a_ref):
        iota = jax.lax.broadcasted_iota(jnp.int32, (n, n), 1)
        acc = jnp.zeros((n, n), F32)
        for kk in range(k):
            col = adj_ref[:, kk : kk + 1]
            acc = acc + (jnp.broadcast_to(col, (n, n)) == iota).astype(F32)
        a_ref[...] = acc * (1.0 / k)

    return pl.pallas_call(
        body,
        out_shape=jax.ShapeDtypeStruct((n, n), F32),
    )(adj)


def kernel(x, adj, hw1, hb1, hw2, hb2, hg, hbeta, rw1, rb1, rw2, rb2, rg,
           rbeta, fw, fb, p1w, p1b, p2w, p2b, p3w, p3b):
    B, C, N = x.shape
    K = adj.shape[1]
    S = hw1.shape[0]
    RES = rw1.shape[0]
    BN = B * N
    TB = 16
    TBN = TB * N
    GRID = B // TB
    MTOT = float(BN)

    A = _build_gather_matrix(adj, N, K)

    x2 = x.transpose(0, 2, 1).reshape(BN, C)

    full = lambda shape: pl.BlockSpec(shape, lambda j: (0,) * len(shape))
    tiled = lambda ch: pl.BlockSpec((TBN, ch), lambda j: (j, 0))

    def conv_pair(h, w1r, w2r, br, ar):
        s_in = jnp.dot(h, w1r[...], preferred_element_type=F32)
        u = jnp.dot(h, w2r[...], preferred_element_type=F32)
        a = ar[...]
        fwd = jnp.concatenate(
            [jnp.dot(a, u[b * N : (b + 1) * N, :], preferred_element_type=F32)
             for b in range(TB)], axis=0)
        return jnp.maximum(s_in + fwd + br[...], 0.0)

    def accum_stats(st_ref, y):
        @pl.when(pl.program_id(0) == 0)
        def _():
            st_ref[...] = jnp.zeros_like(st_ref)
        st_ref[0:1, :] = st_ref[0:1, :] + jnp.sum(y, axis=0, keepdims=True)
        st_ref[1:2, :] = st_ref[1:2, :] + jnp.sum(y * y, axis=0, keepdims=True)

    def norm_affine(st_ref, g_ref, bt_ref):
        mean = st_ref[0:1, :] / MTOT
        var = st_ref[1:2, :] / MTOT - mean * mean
        scale = g_ref[...] * jax.lax.rsqrt(var + EPS)
        shift = bt_ref[...] - mean * scale
        return scale, shift

    def head_body(x_ref, w1r, w2r, br, ar, y_ref, st_ref):
        y = conv_pair(x_ref[...], w1r, w2r, br, ar)
        y_ref[...] = y
        accum_stats(st_ref, y)

    y_prev, st_prev = pl.pallas_call(
        head_body,
        grid=(GRID,),
        in_specs=[tiled(C), full((C, S)), full((C, S)), full((1, S)),
                  full((N, N))],
        out_specs=[tiled(S), full((8, S))],
        out_shape=[jax.ShapeDtypeStruct((BN, S), F32),
                   jax.ShapeDtypeStruct((8, S), F32)],
    )(x2, hw1.T, hw2.T, (hb1 + hb2).reshape(1, S), A)

    def make_res_body(has_resid):
        if has_resid:
            def body(yp_ref, st_ref, hpp_ref, g_ref, bt_ref, w1r, w2r, br,
                     ar, h_ref, y_ref, sto_ref):
                scale, shift = norm_affine(st_ref, g_ref, bt_ref)
                h = yp_ref[...] * scale + shift + hpp_ref[...]
                h_ref[...] = h
                y = conv_pair(h, w1r, w2r, br, ar)
                y_ref[...] = y
                accum_stats(sto_ref, y)
        else:
            def body(yp_ref, st_ref, g_ref, bt_ref, w1r, w2r, br, ar,
                     h_ref, y_ref, sto_ref):
                scale, shift = norm_affine(st_ref, g_ref, bt_ref)
                h = yp_ref[...] * scale + shift
                h_ref[...] = h
                y = conv_pair(h, w1r, w2r, br, ar)
                y_ref[...] = y
                accum_stats(sto_ref, y)
        return body

    hs = []
    h_pp = None
    for i in range(1, RES + 1):
        if i == 1:
            g, bt = hg, hbeta
        else:
            g, bt = rg[i - 2], rbeta[i - 2]
        w1t = rw1[i - 1].T
        w2t = rw2[i - 1].T
        bias = (rb1[i - 1] + rb2[i - 1]).reshape(1, S)
        ins = [y_prev, st_prev]
        specs = [tiled(S), full((8, S))]
        if h_pp is not None:
            ins.append(h_pp)
            specs.append(tiled(S))
        ins += [g.reshape(1, S), bt.reshape(1, S), w1t, w2t, bias, A]
        specs += [full((1, S)), full((1, S)), full((S, S)), full((S, S)),
                  full((1, S)), full((N, N))]
        h_prev, y_prev_new, st_new = pl.pallas_call(
            make_res_body(h_pp is not None),
            grid=(GRID,),
            in_specs=specs,
            out_specs=[tiled(S), tiled(S), full((8, S))],
            out_shape=[jax.ShapeDtypeStruct((BN, S), F32),
                       jax.ShapeDtypeStruct((BN, S), F32),
                       jax.ShapeDtypeStruct((8, S), F32)],
        )(*ins)
        hs.append(h_prev)
        h_pp = h_prev
        y_prev, st_prev = y_prev_new, st_new

    SC = S * (RES + 1)

    def final_body(y7_ref, st_ref, g_ref, bt_ref, h0, h1, h2, h3, h4, h5,
                   h6, fw_ref, fb_ref, pg_ref, ps_ref, p1b_ref, p2_ref,
                   p2b_ref, p3_ref, p3b_ref, o_ref):
        scale, shift = norm_affine(st_ref, g_ref, bt_ref)
        h7 = y7_ref[...] * scale + shift + h6[...]
        hts = [h0[...], h1[...], h2[...], h3[...], h4[...], h5[...],
               h6[...], h7]
        fused = fb_ref[...]
        acc = p1b_ref[...]
        for i in range(RES + 1):
            fused = fused + jnp.dot(hts[i], fw_ref[i * S : (i + 1) * S, :],
                                    preferred_element_type=F32)
            acc = acc + jnp.dot(hts[i], ps_ref[i * S : (i + 1) * S, :],
                                preferred_element_type=F32)
        gs = jnp.concatenate(
            [jnp.max(fused[b * N : (b + 1) * N, :], axis=0, keepdims=True)
             for b in range(TB)], axis=0)
        gc = jnp.dot(gs, pg_ref[...], preferred_element_type=F32)
        gbig = jnp.concatenate(
            [jnp.broadcast_to(gc[b : b + 1, :], (N, gc.shape[1]))
             for b in range(TB)], axis=0)
        y = jnp.maximum(acc + gbig, 0.0)
        y = jnp.maximum(jnp.dot(y, p2_ref[...], preferred_element_type=F32)
                        + p2b_ref[...], 0.0)
        o_ref[...] = jnp.dot(y, p3_ref[...],
                             preferred_element_type=F32) + p3b_ref[...]

    O1 = p1w.shape[0]
    O2 = p2w.shape[0]
    O3 = p3w.shape[0]
    out2 = pl.pallas_call(
        final_body,
        grid=(GRID,),
        in_specs=[tiled(S), full((8, S)), full((1, S)), full((1, S))]
                 + [tiled(S)] * (RES)
                 + [full((SC, O1)), full((1, O1)), full((O1, O1)),
                    full((SC, O1)), full((1, O1)), full((O1, O2)),
                    full((1, O2)), full((O2, O3)), full((1, O3))],
        out_specs=tiled(O3),
        out_shape=jax.ShapeDtypeStruct((BN, O3), F32),
    )(y_prev, st_prev, rg[RES - 1].reshape(1, S),
      rbeta[RES - 1].reshape(1, S), *hs,
      fw.T, fb.reshape(1, O1), p1w[:, :O1].T, p1w[:, O1:].T,
      p1b.reshape(1, O1), p2w.T, p2b.reshape(1, O2), p3w.T,
      p3b.reshape(1, O3))

    return out2.reshape(B, N, O3).transpose(0, 2, 1)


# speedup vs baseline: 9.6035x; 1.0388x over previous
import jax
import jax.numpy as jnp
from jax.experimental import pallas as pl

F32 = jnp.float32
EPS = 1e-5


def _build_gather_matrix(adj, n, k):
    def body(adj_ref, a_ref):
        iota = jax.lax.broadcasted_iota(jnp.int32, (n, n), 1)
        acc = jnp.zeros((n, n), F32)
        for kk in range(k):
            col = adj_ref[:, kk : kk + 1]
            acc = acc + (jnp.broadcast_to(col, (n, n)) == iota).astype(F32)
        a_ref[...] = acc.T * (1.0 / k)

    return pl.pallas_call(
        body,
        out_shape=jax.ShapeDtypeStruct((n, n), F32),
    )(adj)


def kernel(x, adj, hw1, hb1, hw2, hb2, hg, hbeta, rw1, rb1, rw2, rb2, rg,
           rbeta, fw, fb, p1w, p1b, p2w, p2b, p3w, p3b):
    B, C, N = x.shape
    K = adj.shape[1]
    S = hw1.shape[0]
    RES = rw1.shape[0]
    BN = B * N
    TB = 16
    TBN = TB * N
    GRID = B // TB
    MTOT = float(BN)

    A2 = _build_gather_matrix(adj, N, K)

    full = lambda shape: pl.BlockSpec(shape, lambda j: (0,) * len(shape))
    tiled = lambda ch: pl.BlockSpec((ch, TBN), lambda j: (0, j))

    def gather_apply(u, ar):
        a = ar[...]
        return jnp.concatenate(
            [jnp.dot(u[:, b * N : (b + 1) * N], a,
                     preferred_element_type=F32) for b in range(TB)], axis=1)

    def accum_stats(st_ref, y):
        @pl.when(pl.program_id(0) == 0)
        def _():
            st_ref[...] = jnp.zeros_like(st_ref)
        st_ref[:, 0:1] = st_ref[:, 0:1] + jnp.sum(y, axis=1, keepdims=True)
        st_ref[:, 1:2] = st_ref[:, 1:2] + jnp.sum(y * y, axis=1,
                                                  keepdims=True)

    def norm_affine(st_ref, g_ref, bt_ref):
        mean = st_ref[:, 0:1] / MTOT
        var = st_ref[:, 1:2] / MTOT - mean * mean
        scale = g_ref[...] * jax.lax.rsqrt(var + EPS)
        shift = bt_ref[...] - mean * scale
        return scale, shift

    def head_body(x_ref, w1r, w2r, br, ar, y_ref, st_ref):
        w1 = w1r[...]
        w2 = w2r[...]
        u_parts = []
        s_parts = []
        for b in range(TB):
            xb = x_ref[b]
            s_parts.append(jnp.dot(w1, xb, preferred_element_type=F32))
            u_parts.append(jnp.dot(w2, xb, preferred_element_type=F32))
        s_in = jnp.concatenate(s_parts, axis=1)
        u = jnp.concatenate(u_parts, axis=1)
        fwd = gather_apply(u, ar)
        y = jnp.maximum(s_in + fwd + br[...], 0.0)
        y_ref[...] = y
        accum_stats(st_ref, y)

    y_prev, st_prev = pl.pallas_call(
        head_body,
        grid=(GRID,),
        in_specs=[pl.BlockSpec((TB, C, N), lambda j: (j, 0, 0)),
                  full((S, C)), full((S, C)), full((S, 1)), full((N, N))],
        out_specs=[tiled(S), full((S, 8))],
        out_shape=[jax.ShapeDtypeStruct((S, BN), F32),
                   jax.ShapeDtypeStruct((S, 8), F32)],
    )(x, hw1, hw2, (hb1 + hb2).reshape(S, 1), A2)

    def make_res_body(has_resid):
        if has_resid:
            def body(yp_ref, st_ref, hpp_ref, g_ref, bt_ref, wr, br, ar,
                     h_ref, y_ref, sto_ref):
                scale, shift = norm_affine(st_ref, g_ref, bt_ref)
                h = yp_ref[...] * scale + shift + hpp_ref[...]
                h_ref[...] = h
                su = jnp.dot(wr[...], h, preferred_element_type=F32)
                y = jnp.maximum(su[:S] + gather_apply(su[S:], ar) + br[...],
                                0.0)
                y_ref[...] = y
                accum_stats(sto_ref, y)
        else:
            def body(yp_ref, st_ref, g_ref, bt_ref, wr, br, ar,
                     h_ref, y_ref, sto_ref):
                scale, shift = norm_affine(st_ref, g_ref, bt_ref)
                h = yp_ref[...] * scale + shift
                h_ref[...] = h
                su = jnp.dot(wr[...], h, preferred_element_type=F32)
                y = jnp.maximum(su[:S] + gather_apply(su[S:], ar) + br[...],
                                0.0)
                y_ref[...] = y
                accum_stats(sto_ref, y)
        return body

    hs = []
    h_pp = None
    for i in range(1, RES + 1):
        if i == 1:
            g, bt = hg, hbeta
        else:
            g, bt = rg[i - 2], rbeta[i - 2]
        wc = jnp.concatenate([rw1[i - 1], rw2[i - 1]], axis=0)
        bias = (rb1[i - 1] + rb2[i - 1]).reshape(S, 1)
        ins = [y_prev, st_prev]
        specs = [tiled(S), full((S, 8))]
        if h_pp is not None:
            ins.append(h_pp)
            specs.append(tiled(S))
        ins += [g.reshape(S, 1), bt.reshape(S, 1), wc, bias, A2]
        specs += [full((S, 1)), full((S, 1)), full((2 * S, S)),
                  full((S, 1)), full((N, N))]
        h_prev, y_prev_new, st_new = pl.pallas_call(
            make_res_body(h_pp is not None),
            grid=(GRID,),
            in_specs=specs,
            out_specs=[tiled(S), tiled(S), full((S, 8))],
            out_shape=[jax.ShapeDtypeStruct((S, BN), F32),
                       jax.ShapeDtypeStruct((S, BN), F32),
                       jax.ShapeDtypeStruct((S, 8), F32)],
        )(*ins)
        hs.append(h_prev)
        h_pp = h_prev
        y_prev, st_prev = y_prev_new, st_new

    O1 = p1w.shape[0]
    O2 = p2w.shape[0]
    O3 = p3w.shape[0]
    wfs = jnp.concatenate([fw, p1w[:, O1:]], axis=0)

    def final_body(y7_ref, st_ref, g_ref, bt_ref, h0, h1, h2, h3, h4, h5,
                   h6, wf_ref, fb_ref, pg_ref, p1b_ref, p2_ref, p2b_ref,
                   p3_ref, p3b_ref, o_ref):
        scale, shift = norm_affine(st_ref, g_ref, bt_ref)
        h7 = y7_ref[...] * scale + shift + h6[...]
        hts = [h0[...], h1[...], h2[...], h3[...], h4[...], h5[...],
               h6[...], h7]
        fa = jnp.concatenate([fb_ref[...], p1b_ref[...]], axis=0)
        for i in range(RES + 1):
            fa = fa + jnp.dot(wf_ref[:, i * S : (i + 1) * S], hts[i],
                              preferred_element_type=F32)
        fused = fa[:O1]
        acc = fa[O1:]
        gs = jnp.concatenate(
            [jnp.max(fused[:, b * N : (b + 1) * N], axis=1, keepdims=True)
             for b in range(TB)], axis=1)
        gc = jnp.dot(pg_ref[...], gs, preferred_element_type=F32)
        gbig = jnp.concatenate(
            [jnp.broadcast_to(gc[:, b : b + 1], (O1, N))
             for b in range(TB)], axis=1)
        y = jnp.maximum(acc + gbig, 0.0)
        y = jnp.maximum(jnp.dot(p2_ref[...], y, preferred_element_type=F32)
                        + p2b_ref[...], 0.0)
        o_ref[...] = jnp.dot(p3_ref[...], y,
                             preferred_element_type=F32) + p3b_ref[...]

    out2 = pl.pallas_call(
        final_body,
        grid=(GRID,),
        in_specs=[tiled(S), full((S, 8)), full((S, 1)), full((S, 1))]
                 + [tiled(S)] * RES
                 + [full((2 * O1, (RES + 1) * S)), full((O1, 1)),
                    full((O1, O1)), full((O1, 1)), full((O2, O1)),
                    full((O2, 1)), full((O3, O2)), full((O3, 1))],
        out_specs=tiled(O3),
        out_shape=jax.ShapeDtypeStruct((O3, BN), F32),
    )(y_prev, st_prev, rg[RES - 1].reshape(S, 1),
      rbeta[RES - 1].reshape(S, 1), *hs,
      wfs, fb.reshape(O1, 1), p1w[:, :O1], p1b.reshape(O1, 1),
      p2w, p2b.reshape(O2, 1), p3w, p3b.reshape(O3, 1))

    return out2.reshape(O3, B, N).transpose(1, 0, 2)


# speedup vs baseline: 11.6445x; 1.2125x over previous
import jax
import jax.numpy as jnp
from jax.experimental import pallas as pl

F32 = jnp.float32
BF16 = jnp.bfloat16
EPS = 1e-5


def _build_gather_matrix(adj, n, k):
    def body(adj_ref, a_ref):
        iota = jax.lax.broadcasted_iota(jnp.int32, (n, n), 1)
        acc = jnp.zeros((n, n), F32)
        for kk in range(k):
            col = adj_ref[:, kk : kk + 1]
            acc = acc + (jnp.broadcast_to(col, (n, n)) == iota).astype(F32)
        a_ref[...] = acc.T * (1.0 / k)

    return pl.pallas_call(
        body,
        out_shape=jax.ShapeDtypeStruct((n, n), F32),
    )(adj)


def kernel(x, adj, hw1, hb1, hw2, hb2, hg, hbeta, rw1, rb1, rw2, rb2, rg,
           rbeta, fw, fb, p1w, p1b, p2w, p2b, p3w, p3b):
    B, C, N = x.shape
    K = adj.shape[1]
    S = hw1.shape[0]
    RES = rw1.shape[0]
    BN = B * N
    TB = 16
    TBN = TB * N
    GRID = B // TB
    MTOT = float(BN)

    A2 = _build_gather_matrix(adj, N, K)

    full = lambda shape: pl.BlockSpec(shape, lambda j: (0,) * len(shape))
    tiled = lambda ch: pl.BlockSpec((ch, TBN), lambda j: (0, j))

    def gather_apply(u, ar):
        a = ar[...]
        return jnp.concatenate(
            [jnp.dot(u[:, b * N : (b + 1) * N], a,
                     preferred_element_type=F32) for b in range(TB)], axis=1)

    def accum_stats(st_ref, y):
        @pl.when(pl.program_id(0) == 0)
        def _():
            st_ref[...] = jnp.zeros_like(st_ref)
        st_ref[:, 0:1] = st_ref[:, 0:1] + jnp.sum(y, axis=1, keepdims=True)
        st_ref[:, 1:2] = st_ref[:, 1:2] + jnp.sum(y * y, axis=1,
                                                  keepdims=True)

    def norm_affine(st_ref, g_ref, bt_ref):
        mean = st_ref[:, 0:1] / MTOT
        var = st_ref[:, 1:2] / MTOT - mean * mean
        scale = g_ref[...] * jax.lax.rsqrt(var + EPS)
        shift = bt_ref[...] - mean * scale
        return scale, shift

    def head_body(x_ref, w1r, w2r, br, ar, y_ref, st_ref):
        w1 = w1r[...]
        w2 = w2r[...]
        u_parts = []
        s_parts = []
        for b in range(TB):
            xb = x_ref[b]
            s_parts.append(jnp.dot(w1, xb, preferred_element_type=F32))
            u_parts.append(jnp.dot(w2, xb, preferred_element_type=F32))
        s_in = jnp.concatenate(s_parts, axis=1)
        u = jnp.concatenate(u_parts, axis=1)
        fwd = gather_apply(u, ar)
        y = jnp.maximum(s_in + fwd + br[...], 0.0)
        y_ref[...] = y.astype(BF16)
        accum_stats(st_ref, y)

    y_prev, st_prev = pl.pallas_call(
        head_body,
        grid=(GRID,),
        in_specs=[pl.BlockSpec((TB, C, N), lambda j: (j, 0, 0)),
                  full((S, C)), full((S, C)), full((S, 1)), full((N, N))],
        out_specs=[tiled(S), full((S, 8))],
        out_shape=[jax.ShapeDtypeStruct((S, BN), BF16),
                   jax.ShapeDtypeStruct((S, 8), F32)],
    )(x, hw1, hw2, (hb1 + hb2).reshape(S, 1), A2)

    def make_res_body(has_resid):
        if has_resid:
            def body(yp_ref, st_ref, hpp_ref, g_ref, bt_ref, wr, br, ar,
                     h_ref, y_ref, sto_ref):
                scale, shift = norm_affine(st_ref, g_ref, bt_ref)
                h = (yp_ref[...].astype(F32) * scale + shift
                     + hpp_ref[...].astype(F32))
                h_ref[...] = h.astype(BF16)
                su = jnp.dot(wr[...], h, preferred_element_type=F32)
                y = jnp.maximum(su[:S] + gather_apply(su[S:], ar) + br[...],
                                0.0)
                y_ref[...] = y.astype(BF16)
                accum_stats(sto_ref, y)
        else:
            def body(yp_ref, st_ref, g_ref, bt_ref, wr, br, ar,
                     h_ref, y_ref, sto_ref):
                scale, shift = norm_affine(st_ref, g_ref, bt_ref)
                h = yp_ref[...].astype(F32) * scale + shift
                h_ref[...] = h.astype(BF16)
                su = jnp.dot(wr[...], h, preferred_element_type=F32)
                y = jnp.maximum(su[:S] + gather_apply(su[S:], ar) + br[...],
                                0.0)
                y_ref[...] = y.astype(BF16)
                accum_stats(sto_ref, y)
        return body

    hs = []
    h_pp = None
    for i in range(1, RES + 1):
        if i == 1:
            g, bt = hg, hbeta
        else:
            g, bt = rg[i - 2], rbeta[i - 2]
        wc = jnp.concatenate([rw1[i - 1], rw2[i - 1]], axis=0)
        bias = (rb1[i - 1] + rb2[i - 1]).reshape(S, 1)
        ins = [y_prev, st_prev]
        specs = [tiled(S), full((S, 8))]
        if h_pp is not None:
            ins.append(h_pp)
            specs.append(tiled(S))
        ins += [g.reshape(S, 1), bt.reshape(S, 1), wc, bias, A2]
        specs += [full((S, 1)), full((S, 1)), full((2 * S, S)),
                  full((S, 1)), full((N, N))]
        h_prev, y_prev_new, st_new = pl.pallas_call(
            make_res_body(h_pp is not None),
            grid=(GRID,),
            in_specs=specs,
            out_specs=[tiled(S), tiled(S), full((S, 8))],
            out_shape=[jax.ShapeDtypeStruct((S, BN), BF16),
                       jax.ShapeDtypeStruct((S, BN), BF16),
                       jax.ShapeDtypeStruct((S, 8), F32)],
        )(*ins)
        hs.append(h_prev)
        h_pp = h_prev
        y_prev, st_prev = y_prev_new, st_new

    O1 = p1w.shape[0]
    O2 = p2w.shape[0]
    O3 = p3w.shape[0]
    wfs = jnp.concatenate([fw, p1w[:, O1:]], axis=0)

    def final_body(y7_ref, st_ref, g_ref, bt_ref, h0, h1, h2, h3, h4, h5,
                   h6, wf_ref, fb_ref, pg_ref, p1b_ref, p2_ref, p2b_ref,
                   p3_ref, p3b_ref, o_ref):
        scale, shift = norm_affine(st_ref, g_ref, bt_ref)
        h7 = (y7_ref[...].astype(F32) * scale + shift
              + h6[...].astype(F32))
        hts = [h0[...].astype(F32), h1[...].astype(F32),
               h2[...].astype(F32), h3[...].astype(F32),
               h4[...].astype(F32), h5[...].astype(F32),
               h6[...].astype(F32), h7]
        fa = jnp.concatenate([fb_ref[...], p1b_ref[...]], axis=0)
        for i in range(RES + 1):
            fa = fa + jnp.dot(wf_ref[:, i * S : (i + 1) * S], hts[i],
                              preferred_element_type=F32)
        fused = fa[:O1]
        acc = fa[O1:]
        gs = jnp.concatenate(
            [jnp.max(fused[:, b * N : (b + 1) * N], axis=1, keepdims=True)
             for b in range(TB)], axis=1)
        gc = jnp.dot(pg_ref[...], gs, preferred_element_type=F32)
        gbig = jnp.concatenate(
            [jnp.broadcast_to(gc[:, b : b + 1], (O1, N))
             for b in range(TB)], axis=1)
        y = jnp.maximum(acc + gbig, 0.0)
        y = jnp.maximum(jnp.dot(p2_ref[...], y, preferred_element_type=F32)
                        + p2b_ref[...], 0.0)
        o_ref[...] = jnp.dot(p3_ref[...], y,
                             preferred_element_type=F32) + p3b_ref[...]

    out2 = pl.pallas_call(
        final_body,
        grid=(GRID,),
        in_specs=[tiled(S), full((S, 8)), full((S, 1)), full((S, 1))]
                 + [tiled(S)] * RES
                 + [full((2 * O1, (RES + 1) * S)), full((O1, 1)),
                    full((O1, O1)), full((O1, 1)), full((O2, O1)),
                    full((O2, 1)), full((O3, O2)), full((O3, 1))],
        out_specs=tiled(O3),
        out_shape=jax.ShapeDtypeStruct((O3, BN), F32),
    )(y_prev, st_prev, rg[RES - 1].reshape(S, 1),
      rbeta[RES - 1].reshape(S, 1), *hs,
      wfs, fb.reshape(O1, 1), p1w[:, :O1], p1b.reshape(O1, 1),
      p2w, p2b.reshape(O2, 1), p3w, p3b.reshape(O3, 1))

    return out2.reshape(O3, B, N).transpose(1, 0, 2)
